# Initial kernel scaffold; baseline (speedup 1.0000x reference)
import jax
import jax.numpy as jnp
from jax.experimental import pallas as pl


def kernel(x, table, W):
    raise NotImplementedError("write your pallas kernel here")


# speedup vs baseline: 1.3115x; 1.3115x over previous
import functools

import jax
import jax.numpy as jnp
from jax import lax
from jax.experimental import pallas as pl
from jax.experimental.pallas import tpu as pltpu
from jax.experimental.pallas import tpu_sc as plsc

CHUNK = 128


def _gather_rows(idx2d, table, n_chunks_per_w, nw):
    total_chunks, _ = idx2d.shape
    V, D = table.shape
    my_chunks = n_chunks_per_w
    N = total_chunks * CHUNK

    mesh = plsc.VectorSubcoreMesh(core_axis_name="c", subcore_axis_name="s")

    @functools.partial(
        pl.kernel,
        mesh=mesh,
        compiler_params=pltpu.CompilerParams(use_tc_tiling_on_sc=False),
        out_type=jax.ShapeDtypeStruct((N, D), jnp.float32),
        scratch_types=[
            pltpu.VMEM((my_chunks, CHUNK), jnp.int32),
            pltpu.VMEM((CHUNK, D), jnp.float32),
            pltpu.SemaphoreType.DMA,
        ],
    )
    def gather_kernel(idx_hbm, table_hbm, e_hbm, idx_v, rows_v, sem):
        wid = lax.axis_index("s") * 2 + lax.axis_index("c")
        chunk_base = wid * my_chunks
        pltpu.sync_copy(idx_hbm.at[pl.ds(chunk_base, my_chunks)], idx_v)

        def body(j, carry):
            pltpu.async_copy(table_hbm.at[idx_v.at[j]], rows_v, sem).wait()
            row0 = (chunk_base + j) * CHUNK
            pltpu.sync_copy(rows_v, e_hbm.at[pl.ds(row0, CHUNK)])
            return carry

        lax.fori_loop(0, my_chunks, body, 0)

    return gather_kernel(idx2d, table)


def _mm_body(e_ref, w_ref, o_ref):
    o_ref[...] = lax.dot_general(
        e_ref[...], w_ref[...],
        dimension_numbers=(((1,), (1,)), ((), ())),
        preferred_element_type=jnp.float32,
    )


def _project(e, W, blk):
    N, D = e.shape
    H, _ = W.shape
    return pl.pallas_call(
        _mm_body,
        grid=(N // blk,),
        in_specs=[
            pl.BlockSpec((blk, D), lambda i: (i, 0)),
            pl.BlockSpec((H, D), lambda i: (0, 0)),
        ],
        out_specs=pl.BlockSpec((blk, H), lambda i: (i, 0)),
        out_shape=jax.ShapeDtypeStruct((N, H), jnp.float32),
    )(e, W)


def kernel(x, table, W):
    B, L = x.shape
    N = B * L
    H = W.shape[0]
    nw = 32
    total_chunks = N // CHUNK
    idx2d = x.reshape(total_chunks, CHUNK).astype(jnp.int32)
    e = _gather_rows(idx2d, table, total_chunks // nw, nw)
    out = _project(e, W, blk=512)
    return out.reshape(B, L, H)


# speedup vs baseline: 1.8914x; 1.4422x over previous
import functools

import jax
import jax.numpy as jnp
from jax import lax
from jax.experimental import pallas as pl
from jax.experimental.pallas import tpu as pltpu
from jax.experimental.pallas import tpu_sc as plsc

CHUNK = 128
NW = 32


def _gather_rows(idx2d, table):
    total_chunks, _ = idx2d.shape
    V, D = table.shape
    half_chunks = total_chunks // 2
    mc = half_chunks // NW
    M = half_chunks * CHUNK

    mesh = plsc.VectorSubcoreMesh(core_axis_name="c", subcore_axis_name="s")

    @functools.partial(
        pl.kernel,
        mesh=mesh,
        compiler_params=pltpu.CompilerParams(use_tc_tiling_on_sc=False),
        out_type=jax.ShapeDtypeStruct((M, 2 * D), jnp.float32),
        scratch_types=[
            pltpu.VMEM((2 * mc, CHUNK), jnp.int32),
            pltpu.VMEM((CHUNK, D), jnp.float32),
            pltpu.VMEM((CHUNK, D), jnp.float32),
            pltpu.SemaphoreType.DMA,
            pltpu.SemaphoreType.DMA,
        ],
    )
    def gather_kernel(idx_hbm, table_hbm, e_hbm, idx_v, rows_a, rows_b,
                      sem_a, sem_b):
        wid = lax.axis_index("s") * 2 + lax.axis_index("c")
        chunk_base = wid * mc
        pltpu.sync_copy(idx_hbm.at[pl.ds(chunk_base, mc)],
                        idx_v.at[pl.ds(0, mc)])
        pltpu.sync_copy(idx_hbm.at[pl.ds(half_chunks + chunk_base, mc)],
                        idx_v.at[pl.ds(mc, mc)])

        def body(j, carry):
            cp_a = pltpu.async_copy(table_hbm.at[idx_v.at[j]], rows_a, sem_a)
            cp_b = pltpu.async_copy(table_hbm.at[idx_v.at[mc + j]], rows_b,
                                    sem_b)
            cp_a.wait()
            cp_b.wait()
            row0 = (chunk_base + j) * CHUNK
            pltpu.sync_copy(rows_a, e_hbm.at[pl.ds(row0, CHUNK), pl.ds(0, D)])
            pltpu.sync_copy(rows_b, e_hbm.at[pl.ds(row0, CHUNK), pl.ds(D, D)])
            return carry

        lax.fori_loop(0, mc, body, 0)

    return gather_kernel(idx2d, table)


def _mm_body(h, e_ref, w_ref, o_ref):
    d = lax.dot_general(
        e_ref[...], w_ref[...],
        dimension_numbers=(((1,), (1,)), ((), ())),
        preferred_element_type=jnp.float32,
    )
    o_ref[0] = d[:, :h]
    o_ref[1] = d[:, h:]


def _project(e2, W2, blk):
    M, K = e2.shape
    H = W2.shape[0] // 2
    return pl.pallas_call(
        functools.partial(_mm_body, H),
        grid=(M // blk,),
        in_specs=[
            pl.BlockSpec((blk, K), lambda i: (i, 0)),
            pl.BlockSpec((2 * H, K), lambda i: (0, 0)),
        ],
        out_specs=pl.BlockSpec((2, blk, H), lambda i: (0, i, 0)),
        out_shape=jax.ShapeDtypeStruct((2, M, H), jnp.float32),
    )(e2, W2)


def kernel(x, table, W):
    B, L = x.shape
    N = B * L
    H, D = W.shape
    total_chunks = N // CHUNK
    idx2d = x.reshape(total_chunks, CHUNK).astype(jnp.int32)
    e2 = _gather_rows(idx2d, table)
    W2 = jnp.zeros((2 * H, 2 * D), W.dtype)
    W2 = W2.at[:H, :D].set(W).at[H:, D:].set(W)
    out2 = _project(e2, W2, blk=512)
    return out2.reshape(B, L, H)


# speedup vs baseline: 2.0886x; 1.1043x over previous
import functools

import jax
import jax.numpy as jnp
from jax import lax
from jax.experimental import pallas as pl
from jax.experimental.pallas import tpu as pltpu
from jax.experimental.pallas import tpu_sc as plsc

CHUNK = 128
NW = 32


def _gather_rows(idx2d, table):
    total_chunks, _ = idx2d.shape
    V, D = table.shape
    half_chunks = total_chunks // 2
    mc = half_chunks // NW
    M = half_chunks * CHUNK

    mesh = plsc.VectorSubcoreMesh(core_axis_name="c", subcore_axis_name="s")

    @functools.partial(
        pl.kernel,
        mesh=mesh,
        compiler_params=pltpu.CompilerParams(use_tc_tiling_on_sc=False),
        out_type=jax.ShapeDtypeStruct((M, 2 * D), jnp.float32),
        scratch_types=[
            pltpu.VMEM((2 * mc, CHUNK), jnp.int32),
            pltpu.VMEM((CHUNK, D), jnp.float32),
            pltpu.VMEM((CHUNK, D), jnp.float32),
            pltpu.SemaphoreType.DMA,
            pltpu.SemaphoreType.DMA,
        ],
    )
    def gather_kernel(idx_hbm, table_hbm, e_hbm, idx_v, rows_a, rows_b,
                      sem_a, sem_b):
        wid = lax.axis_index("s") * 2 + lax.axis_index("c")
        chunk_base = wid * mc
        pltpu.sync_copy(idx_hbm.at[pl.ds(chunk_base, mc)],
                        idx_v.at[pl.ds(0, mc)])
        pltpu.sync_copy(idx_hbm.at[pl.ds(half_chunks + chunk_base, mc)],
                        idx_v.at[pl.ds(mc, mc)])

        def body(j, carry):
            cp_a = pltpu.async_copy(table_hbm.at[idx_v.at[j]], rows_a, sem_a)
            cp_b = pltpu.async_copy(table_hbm.at[idx_v.at[mc + j]], rows_b,
                                    sem_b)
            cp_a.wait()
            cp_b.wait()
            row0 = (chunk_base + j) * CHUNK
            pltpu.sync_copy(rows_a, e_hbm.at[pl.ds(row0, CHUNK), pl.ds(0, D)])
            pltpu.sync_copy(rows_b, e_hbm.at[pl.ds(row0, CHUNK), pl.ds(D, D)])
            return carry

        lax.fori_loop(0, mc, body, 0)

    return gather_kernel(idx2d, table)


def _mm_body(h, e_ref, w_ref, o_ref):
    d = lax.dot_general(
        e_ref[...], w_ref[...],
        dimension_numbers=(((1,), (1,)), ((), ())),
        preferred_element_type=jnp.float32,
    )
    o_ref[0] = d[:, :h]
    o_ref[1] = d[:, h:]


def _project(e2, W2, blk):
    M, K = e2.shape
    H = W2.shape[0] // 2
    return pl.pallas_call(
        functools.partial(_mm_body, H),
        grid=(M // blk,),
        in_specs=[
            pl.BlockSpec((blk, K), lambda i: (i, 0)),
            pl.BlockSpec((2 * H, K), lambda i: (0, 0)),
        ],
        out_specs=pl.BlockSpec((2, blk, H), lambda i: (0, i, 0)),
        out_shape=jax.ShapeDtypeStruct((2, M, H), jnp.float32),
    )(e2, W2)


def kernel(x, table, W):
    B, L = x.shape
    N = B * L
    H, D = W.shape
    total_chunks = N // CHUNK
    idx2d = x.reshape(total_chunks, CHUNK).astype(jnp.int32)
    e2 = _gather_rows(idx2d, table)
    W2 = jnp.zeros((2 * H, 2 * D), W.dtype)
    W2 = W2.at[:H, :D].set(W).at[H:, D:].set(W)
    out2 = _project(e2, W2, blk=1024)
    return out2.reshape(B, L, H)


# speedup vs baseline: 2.1139x; 1.0121x over previous
import functools

import jax
import jax.numpy as jnp
from jax import lax
from jax.experimental import pallas as pl
from jax.experimental.pallas import tpu as pltpu
from jax.experimental.pallas import tpu_sc as plsc

CHUNK = 128
NW = 32


def _gather_rows(idx2d, table):
    total_chunks, _ = idx2d.shape
    V, D = table.shape
    half_chunks = total_chunks // 2
    mc = half_chunks // NW
    M = half_chunks * CHUNK

    mesh = plsc.VectorSubcoreMesh(core_axis_name="c", subcore_axis_name="s")

    @functools.partial(
        pl.kernel,
        mesh=mesh,
        compiler_params=pltpu.CompilerParams(use_tc_tiling_on_sc=False),
        out_type=jax.ShapeDtypeStruct((M, 2 * D), jnp.float32),
        scratch_types=[
            pltpu.VMEM((2 * mc, CHUNK), jnp.int32),
            pltpu.VMEM((CHUNK, D), jnp.float32),
            pltpu.VMEM((CHUNK, D), jnp.float32),
            pltpu.SemaphoreType.DMA,
            pltpu.SemaphoreType.DMA,
        ],
    )
    def gather_kernel(idx_hbm, table_hbm, e_hbm, idx_v, rows_a, rows_b,
                      sem_a, sem_b):
        wid = lax.axis_index("s") * 2 + lax.axis_index("c")
        chunk_base = wid * mc
        pltpu.sync_copy(idx_hbm.at[pl.ds(chunk_base, mc)],
                        idx_v.at[pl.ds(0, mc)])
        pltpu.sync_copy(idx_hbm.at[pl.ds(half_chunks + chunk_base, mc)],
                        idx_v.at[pl.ds(mc, mc)])

        def body(j, carry):
            cp_a = pltpu.async_copy(table_hbm.at[idx_v.at[j]], rows_a, sem_a)
            cp_b = pltpu.async_copy(table_hbm.at[idx_v.at[mc + j]], rows_b,
                                    sem_b)
            cp_a.wait()
            cp_b.wait()
            row0 = (chunk_base + j) * CHUNK
            pltpu.sync_copy(rows_a, e_hbm.at[pl.ds(row0, CHUNK), pl.ds(0, D)])
            pltpu.sync_copy(rows_b, e_hbm.at[pl.ds(row0, CHUNK), pl.ds(D, D)])
            return carry

        lax.fori_loop(0, mc, body, 0)

    return gather_kernel(idx2d, table)


def _mm_body(h, e_ref, w_ref, o_ref):
    d = lax.dot_general(
        e_ref[...], w_ref[...],
        dimension_numbers=(((1,), (1,)), ((), ())),
        preferred_element_type=jnp.float32,
    )
    o_ref[0] = d[:, :h]
    o_ref[1] = d[:, h:]


def _project(e2, W2, blk):
    M, K = e2.shape
    H = W2.shape[0] // 2
    return pl.pallas_call(
        functools.partial(_mm_body, H),
        grid=(M // blk,),
        in_specs=[
            pl.BlockSpec((blk, K), lambda i: (i, 0)),
            pl.BlockSpec((2 * H, K), lambda i: (0, 0)),
        ],
        out_specs=pl.BlockSpec((2, blk, H), lambda i: (0, i, 0)),
        out_shape=jax.ShapeDtypeStruct((2, M, H), jnp.float32),
    )(e2, W2)


def kernel(x, table, W):
    B, L = x.shape
    N = B * L
    H, D = W.shape
    total_chunks = N // CHUNK
    idx2d = x.reshape(total_chunks, CHUNK).astype(jnp.int32)
    e2 = _gather_rows(idx2d, table)
    W2 = jnp.zeros((2 * H, 2 * D), W.dtype)
    W2 = W2.at[:H, :D].set(W).at[H:, D:].set(W)
    out2 = _project(e2, W2, blk=2048)
    return out2.reshape(B, L, H)
